# baseline (speedup 1.0000x reference)
import functools

import jax
import jax.numpy as jnp
from jax import lax
from jax.experimental import pallas as pl
from jax.experimental.pallas import tpu as pltpu
from jax.experimental.pallas import tpu_sc as plsc

_SUB = 128
_LANES = 16


def _gather_pairs_chunked(table, inv_lin, n_pairs, embed, n):
    info = plsc.get_sparse_core_info()
    nc, ns = info.num_cores, info.num_subcores
    nw = nc * ns
    pairs_per_w = n_pairs // nw
    assert pairs_per_w == 2 * _SUB
    rows_per_j = 2 * n_pairs // _SUB
    w_rows = pairs_per_w // _SUB

    mesh = plsc.VectorSubcoreMesh(core_axis_name="c", subcore_axis_name="s")

    @functools.partial(
        pl.kernel,
        mesh=mesh,
        compiler_params=pltpu.CompilerParams(use_tc_tiling_on_sc=False,
                                             needs_layout_passes=False),
        out_type=jax.ShapeDtypeStruct((n, n_pairs, 2 * embed), jnp.float32),
        scratch_types=[
            pltpu.VMEM((2, n, w_rows, _SUB), jnp.int32),
            pltpu.VMEM((2, 2, pairs_per_w, embed), jnp.float32),
            pltpu.SemaphoreType.DMA,
            pltpu.SemaphoreType.DMA,
            pltpu.SemaphoreType.DMA,
            pltpu.SemaphoreType.DMA,
        ],
    )
    def gather_kernel(table_hbm, inv_hbm, out_hbm, idx_v, bufs,
                      g0, g1, w0, w1):
        wid = lax.axis_index("s") * nc + lax.axis_index("c")
        p0 = wid * pairs_per_w
        gsem = (g0, g1)
        wsem = (w0, w1)
        idx_copies = []
        for b01 in (0, 1):
            for j in range(n):
                src = (j * rows_per_j + b01 * (rows_per_j // 2)
                       + w_rows * wid)
                idx_copies.append(pltpu.async_copy(
                    inv_hbm.at[pl.ds(src, w_rows)], idx_v.at[b01, j], g0))
        for c in idx_copies:
            c.wait()

        def fire(r):
            cur = r % 2
            copies = []
            for par in (0, 1):
                b01, j = divmod(2 * r + par, n)
                for h in (0, 1):
                    copies.append(pltpu.async_copy(
                        table_hbm.at[idx_v.at[b01, j, h]],
                        bufs.at[cur, par, pl.ds(h * _SUB, _SUB)],
                        gsem[cur],
                    ))
            return copies

        def put(r):
            cur = r % 2
            return [
                pltpu.async_copy(
                    bufs.at[cur, par],
                    out_hbm.at[r, pl.ds(p0, pairs_per_w),
                               pl.ds(par * embed, embed)],
                    wsem[cur],
                )
                for par in (0, 1)
            ]

        pending_g = fire(0)
        pending_w = [None, None]
        for r in range(n):
            cur = r % 2
            if r + 1 < n:
                if pending_w[1 - cur] is not None:
                    for c in pending_w[1 - cur]:
                        c.wait()
                next_g = fire(r + 1)
            for c in pending_g:
                c.wait()
            if r + 1 < n:
                pending_g = next_g
            pending_w[cur] = put(r)
        for side in (0, 1):
            for c in pending_w[side]:
                c.wait()

    return gather_kernel(table, inv_lin)


def _unpad_body(x_ref, o_ref):
    x = x_ref[...]
    x3 = x.reshape(x.shape[0] // 2, 2, x.shape[1])
    o_ref[...] = jnp.concatenate([x3[:, 0, :], x3[:, 1, :]], axis=1)


def _pack_rows(table, bm=4000):
    m, d = table.shape
    return pl.pallas_call(
        _unpad_body,
        grid=(m // bm,),
        in_specs=[pl.BlockSpec((bm, d), lambda i: (i, 0))],
        out_specs=pl.BlockSpec((bm // 2, 2 * d), lambda i: (i, 0)),
        out_shape=jax.ShapeDtypeStruct((m // 2, 2 * d), jnp.float32),
    )(table)


def _mm_body(x_ref, w_ref, b_ref, o_ref):
    n_chunks = x_ref.shape[0]
    acc = None
    for r in range(n_chunks):
        xr = x_ref[r].astype(jnp.bfloat16)
        d = jnp.dot(xr, w_ref[r], preferred_element_type=jnp.float32)
        acc = d if acc is None else acc + d
    acc += b_ref[...]
    half = o_ref.shape[2]
    o_ref[0] = acc[:, :half]
    o_ref[1] = acc[:, half:]


def _matmul_chunks(x9, w9, b2, block_m):
    n_chunks, m, k = x9.shape
    n = w9.shape[2]
    return pl.pallas_call(
        _mm_body,
        grid=(m // block_m,),
        in_specs=[
            pl.BlockSpec((n_chunks, block_m, k), lambda i: (0, i, 0)),
            pl.BlockSpec((n_chunks, k, n), lambda i: (0, 0, 0)),
            pl.BlockSpec((1, n), lambda i: (0, 0)),
        ],
        out_specs=pl.BlockSpec((2, block_m, n // 2), lambda i: (0, i, 0)),
        out_shape=jax.ShapeDtypeStruct((2, m, n // 2), jnp.float32),
    )(x9, w9, b2)


def kernel(inventory, node_embeds, W, b):
    batch, n = inventory.shape
    vocab, embed = node_embeds.shape
    out_dim = W.shape[0]
    n_pairs = batch // 2

    inv_jmaj = inventory.T.reshape(batch * n // _SUB, _SUB)
    table_lin = _pack_rows(node_embeds).reshape(vocab, embed)
    x9 = _gather_pairs_chunked(table_lin, inv_jmaj, n_pairs, embed, n)

    wt = W.T
    blocks = []
    for r in range(n):
        cols = []
        for u in (2 * r, 2 * r + 1):
            b01, j = divmod(u, n)
            piece = wt[j * embed:(j + 1) * embed]
            zero = jnp.zeros_like(piece)
            half = (jnp.concatenate([piece, zero], axis=1) if b01 == 0
                    else jnp.concatenate([zero, piece], axis=1))
            cols.append(half)
        blocks.append(jnp.concatenate(cols, axis=0))
    w9 = jnp.stack(blocks).astype(jnp.bfloat16)
    b2 = jnp.concatenate([b, b]).reshape(1, 2 * out_dim)

    out3 = _matmul_chunks(x9, w9, b2, block_m=1024)
    return out3.reshape(batch, out_dim)


# speedup vs baseline: 1.1383x; 1.1383x over previous
import functools

import jax
import jax.numpy as jnp
from jax import lax
from jax.experimental import pallas as pl
from jax.experimental.pallas import tpu as pltpu
from jax.experimental.pallas import tpu_sc as plsc

_SUB = 128
_LANES = 16


def _gather_pairs_chunked(table, inv_lin, n_pairs, embed, n):
    info = plsc.get_sparse_core_info()
    nc, ns = info.num_cores, info.num_subcores
    nw = nc * ns
    pairs_per_w = n_pairs // nw
    assert pairs_per_w == 2 * _SUB
    rows_per_j = 2 * n_pairs // _SUB
    w_rows = pairs_per_w // _SUB

    mesh = plsc.VectorSubcoreMesh(core_axis_name="c", subcore_axis_name="s")

    @functools.partial(
        pl.kernel,
        mesh=mesh,
        compiler_params=pltpu.CompilerParams(use_tc_tiling_on_sc=False,
                                             needs_layout_passes=False),
        out_type=jax.ShapeDtypeStruct((n, n_pairs, 2 * embed), jnp.float32),
        scratch_types=[
            pltpu.VMEM((2, n, w_rows, _SUB), jnp.int32),
            pltpu.VMEM((2, 2, pairs_per_w, embed), jnp.float32),
            pltpu.SemaphoreType.DMA,
            pltpu.SemaphoreType.DMA,
            pltpu.SemaphoreType.DMA,
            pltpu.SemaphoreType.DMA,
        ],
    )
    def gather_kernel(table_hbm, inv_hbm, out_hbm, idx_v, bufs,
                      g0, g1, w0, w1):
        wid = lax.axis_index("s") * nc + lax.axis_index("c")
        p0 = wid * pairs_per_w
        gsem = (g0, g1)
        wsem = (w0, w1)
        idx_copies = []
        for b01 in (0, 1):
            for j in range(n):
                src = (j * rows_per_j + b01 * (rows_per_j // 2)
                       + w_rows * wid)
                idx_copies.append(pltpu.async_copy(
                    inv_hbm.at[pl.ds(src, w_rows)], idx_v.at[b01, j], g0))
        for c in idx_copies:
            c.wait()

        def fire(r):
            cur = r % 2
            copies = []
            for par in (0, 1):
                b01, j = divmod(2 * r + par, n)
                for h in (0, 1):
                    copies.append(pltpu.async_copy(
                        table_hbm.at[idx_v.at[b01, j, h]],
                        bufs.at[cur, par, pl.ds(h * _SUB, _SUB)],
                        gsem[cur],
                    ))
            return copies

        def put(r):
            cur = r % 2
            return [
                pltpu.async_copy(
                    bufs.at[cur, par],
                    out_hbm.at[r, pl.ds(p0, pairs_per_w),
                               pl.ds(par * embed, embed)],
                    wsem[cur],
                )
                for par in (0, 1)
            ]

        pending_g = fire(0)
        pending_w = [None, None]
        for r in range(n):
            cur = r % 2
            if r + 1 < n:
                if pending_w[1 - cur] is not None:
                    for c in pending_w[1 - cur]:
                        c.wait()
                next_g = fire(r + 1)
            for c in pending_g:
                c.wait()
            if r + 1 < n:
                pending_g = next_g
            pending_w[cur] = put(r)
        for side in (0, 1):
            for c in pending_w[side]:
                c.wait()

    return gather_kernel(table, inv_lin)


def _unpad_body(x_ref, o_ref):
    x = x_ref[...]
    x3 = x.reshape(x.shape[0] // 2, 2, x.shape[1])
    o_ref[...] = jnp.concatenate([x3[:, 0, :], x3[:, 1, :]], axis=1)


def _pack_rows(table, bm=4000):
    m, d = table.shape
    return pl.pallas_call(
        _unpad_body,
        grid=(m // bm,),
        in_specs=[pl.BlockSpec((bm, d), lambda i: (i, 0))],
        out_specs=pl.BlockSpec((bm // 2, 2 * d), lambda i: (i, 0)),
        out_shape=jax.ShapeDtypeStruct((m // 2, 2 * d), jnp.float32),
    )(table)


def _mm_body(x_ref, w_ref, b_ref, o_ref):
    n_chunks = x_ref.shape[0]
    acc = None
    for r in range(n_chunks):
        xr = x_ref[r].astype(jnp.bfloat16)
        d = jnp.dot(xr, w_ref[r], preferred_element_type=jnp.float32)
        acc = d if acc is None else acc + d
    acc += b_ref[...]
    half = o_ref.shape[2]
    o_ref[0] = acc[:, :half]
    o_ref[1] = acc[:, half:]


def _matmul_chunks(x9, w9, b2, block_m):
    n_chunks, m, k = x9.shape
    n = w9.shape[2]
    return pl.pallas_call(
        _mm_body,
        grid=(m // block_m,),
        in_specs=[
            pl.BlockSpec((n_chunks, block_m, k), lambda i: (0, i, 0)),
            pl.BlockSpec((n_chunks, k, n), lambda i: (0, 0, 0)),
            pl.BlockSpec((1, n), lambda i: (0, 0)),
        ],
        out_specs=pl.BlockSpec((2, block_m, n // 2), lambda i: (0, i, 0)),
        out_shape=jax.ShapeDtypeStruct((2, m, n // 2), jnp.float32),
    )(x9, w9, b2)


def kernel(inventory, node_embeds, W, b):
    batch, n = inventory.shape
    vocab, embed = node_embeds.shape
    out_dim = W.shape[0]
    n_pairs = batch // 2

    inv_jmaj = inventory.T.reshape(batch * n // _SUB, _SUB)
    x9 = _gather_pairs_chunked(node_embeds, inv_jmaj, n_pairs, embed, n)

    wt = W.T
    blocks = []
    for r in range(n):
        cols = []
        for u in (2 * r, 2 * r + 1):
            b01, j = divmod(u, n)
            piece = wt[j * embed:(j + 1) * embed]
            zero = jnp.zeros_like(piece)
            half = (jnp.concatenate([piece, zero], axis=1) if b01 == 0
                    else jnp.concatenate([zero, piece], axis=1))
            cols.append(half)
        blocks.append(jnp.concatenate(cols, axis=0))
    w9 = jnp.stack(blocks).astype(jnp.bfloat16)
    b2 = jnp.concatenate([b, b]).reshape(1, 2 * out_dim)

    out3 = _matmul_chunks(x9, w9, b2, block_m=1024)
    return out3.reshape(batch, out_dim)


# speedup vs baseline: 1.1455x; 1.0063x over previous
import functools

import jax
import jax.numpy as jnp
from jax import lax
from jax.experimental import pallas as pl
from jax.experimental.pallas import tpu as pltpu
from jax.experimental.pallas import tpu_sc as plsc

_SUB = 128
_LANES = 16


def _gather_pairs_chunked(table, inv_lin, n_pairs, embed, n):
    info = plsc.get_sparse_core_info()
    nc, ns = info.num_cores, info.num_subcores
    nw = nc * ns
    pairs_per_w = n_pairs // nw
    assert pairs_per_w == 2 * _SUB
    rows_per_j = 2 * n_pairs // _SUB
    w_rows = pairs_per_w // _SUB

    mesh = plsc.VectorSubcoreMesh(core_axis_name="c", subcore_axis_name="s")

    @functools.partial(
        pl.kernel,
        mesh=mesh,
        compiler_params=pltpu.CompilerParams(use_tc_tiling_on_sc=False,
                                             needs_layout_passes=False),
        out_type=jax.ShapeDtypeStruct((n, n_pairs, 2 * embed), jnp.float32),
        scratch_types=[
            pltpu.VMEM((2, n, w_rows, _SUB), jnp.int32),
            pltpu.VMEM((2, 2, pairs_per_w, embed), jnp.float32),
            pltpu.SemaphoreType.DMA,
            pltpu.SemaphoreType.DMA,
            pltpu.SemaphoreType.DMA,
            pltpu.SemaphoreType.DMA,
        ],
    )
    def gather_kernel(table_hbm, inv_hbm, out_hbm, idx_v, bufs,
                      g0, g1, w0, w1):
        wid = lax.axis_index("s") * nc + lax.axis_index("c")
        p0 = wid * pairs_per_w
        gsem = (g0, g1)
        wsem = (w0, w1)
        idx_copies = []
        for b01 in (0, 1):
            for j in range(n):
                src = (j * rows_per_j + b01 * (rows_per_j // 2)
                       + w_rows * wid)
                idx_copies.append(pltpu.async_copy(
                    inv_hbm.at[pl.ds(src, w_rows)], idx_v.at[b01, j], g0))
        for c in idx_copies:
            c.wait()

        def fire(r):
            cur = r % 2
            copies = []
            for par in (0, 1):
                b01, j = divmod(2 * r + par, n)
                for h in (0, 1):
                    copies.append(pltpu.async_copy(
                        table_hbm.at[idx_v.at[b01, j, h]],
                        bufs.at[cur, par, pl.ds(h * _SUB, _SUB)],
                        gsem[cur],
                    ))
            return copies

        def put(r):
            cur = r % 2
            return [
                pltpu.async_copy(
                    bufs.at[cur, par],
                    out_hbm.at[r, pl.ds(p0, pairs_per_w),
                               pl.ds(par * embed, embed)],
                    wsem[cur],
                )
                for par in (0, 1)
            ]

        pending_g = fire(0)
        pending_w = [None, None]
        for r in range(n):
            cur = r % 2
            if r + 1 < n:
                if pending_w[1 - cur] is not None:
                    for c in pending_w[1 - cur]:
                        c.wait()
                next_g = fire(r + 1)
            for c in pending_g:
                c.wait()
            if r + 1 < n:
                pending_g = next_g
            pending_w[cur] = put(r)
        for side in (0, 1):
            for c in pending_w[side]:
                c.wait()

    return gather_kernel(table, inv_lin)


def _mm_body(x_ref, w_ref, b_ref, o_ref):
    n_chunks = x_ref.shape[0]
    acc = None
    for r in range(n_chunks):
        xr = x_ref[r].astype(jnp.bfloat16)
        d = jnp.dot(xr, w_ref[r], preferred_element_type=jnp.float32)
        acc = d if acc is None else acc + d
    acc += b_ref[...]
    half = o_ref.shape[2]
    o_ref[0] = acc[:, :half]
    o_ref[1] = acc[:, half:]


def _matmul_chunks(x9, w9, b2, block_m):
    n_chunks, m, k = x9.shape
    n = w9.shape[2]
    return pl.pallas_call(
        _mm_body,
        grid=(m // block_m,),
        in_specs=[
            pl.BlockSpec((n_chunks, block_m, k), lambda i: (0, i, 0)),
            pl.BlockSpec((n_chunks, k, n), lambda i: (0, 0, 0)),
            pl.BlockSpec((1, n), lambda i: (0, 0)),
        ],
        out_specs=pl.BlockSpec((2, block_m, n // 2), lambda i: (0, i, 0)),
        out_shape=jax.ShapeDtypeStruct((2, m, n // 2), jnp.float32),
    )(x9, w9, b2)


def kernel(inventory, node_embeds, W, b):
    batch, n = inventory.shape
    vocab, embed = node_embeds.shape
    out_dim = W.shape[0]
    n_pairs = batch // 2

    inv_jmaj = inventory.T.reshape(batch * n // _SUB, _SUB)
    x9 = _gather_pairs_chunked(node_embeds, inv_jmaj, n_pairs, embed, n)

    wt = W.T
    blocks = []
    for r in range(n):
        cols = []
        for u in (2 * r, 2 * r + 1):
            b01, j = divmod(u, n)
            piece = wt[j * embed:(j + 1) * embed]
            zero = jnp.zeros_like(piece)
            half = (jnp.concatenate([piece, zero], axis=1) if b01 == 0
                    else jnp.concatenate([zero, piece], axis=1))
            cols.append(half)
        blocks.append(jnp.concatenate(cols, axis=0))
    w9 = jnp.stack(blocks).astype(jnp.bfloat16)
    b2 = jnp.concatenate([b, b]).reshape(1, 2 * out_dim)

    out3 = _matmul_chunks(x9, w9, b2, block_m=2048)
    return out3.reshape(batch, out_dim)
